# baseline (speedup 1.0000x reference)
import functools

import jax
import jax.numpy as jnp
from jax import lax
from jax.experimental import pallas as pl
from jax.experimental.pallas import tpu as pltpu
from jax.experimental.pallas import tpu_sc as plsc

_LANES = 128


def _make_segsum(N, E, D):
    DC = D // 2
    NCH = E // _LANES
    NTILES = 16
    base_per_tile = NCH // NTILES
    rem = NCH - base_per_tile * NTILES
    NBUF = 3
    npairs = base_per_tile // NBUF
    assert npairs * NBUF == base_per_tile
    rows_main = (N // NTILES) // 8 * 8
    rows_tail = N - rows_main * NTILES

    mesh = plsc.VectorSubcoreMesh(core_axis_name="c", subcore_axis_name="s")

    @functools.partial(
        pl.kernel,
        mesh=mesh,
        out_type=jax.ShapeDtypeStruct((N, D), jnp.float32),
        scratch_types=[
            pltpu.VMEM((NBUF, _LANES), jnp.int32),
            pltpu.VMEM((NBUF * _LANES, DC), jnp.float32),
            pltpu.VMEM_SHARED((N, DC), jnp.float32),
            pltpu.SemaphoreType.DMA,
            pltpu.SemaphoreType.DMA,
            pltpu.SemaphoreType.DMA,
        ],
    )
    def segsum(h_hbm, dst_hbm, zeros_hbm, out_hbm, idx_v, hbuf, acc_sh,
               sem0, sem1, sem2):
        cc = lax.axis_index("c")
        sid = lax.axis_index("s")
        col0 = cc * DC
        r0 = sid * rows_main
        sems = (sem0, sem1, sem2)

        pltpu.sync_copy(zeros_hbm, acc_sh.at[pl.ds(r0, rows_main)])

        @pl.when(sid == NTILES - 1)
        def _():
            pltpu.sync_copy(
                zeros_hbm.at[pl.ds(0, rows_tail)],
                acc_sh.at[pl.ds(NTILES * rows_main, rows_tail)],
            )

        plsc.subcore_barrier()

        chunk_base = sid * base_per_tile
        last_chunk = chunk_base + base_per_tile - 1

        def copies(b, ch):
            e0 = ch * _LANES
            return (
                pltpu.make_async_copy(
                    h_hbm.at[pl.ds(e0, _LANES), pl.ds(col0, DC)],
                    hbuf.at[pl.ds(b * _LANES, _LANES)],
                    sems[b],
                ),
                pltpu.make_async_copy(
                    dst_hbm.at[pl.ds(e0, _LANES)], idx_v.at[b], sems[b]
                ),
            )

        def start_load(b, ch):
            for c in copies(b, ch):
                c.start()

        def wait_load(b, ch):
            for c in copies(b, ch):
                c.wait()

        for b in range(NBUF):
            start_load(b, chunk_base + b)

        def pbody(p, carry):
            for b in range(NBUF):
                ch = chunk_base + p * NBUF + b
                wait_load(b, ch)
                pltpu.sync_copy(
                    hbuf.at[pl.ds(b * _LANES, _LANES)],
                    acc_sh.at[idx_v.at[b]],
                    add=True,
                )
                start_load(b, jnp.minimum(ch + NBUF, last_chunk))
            return carry

        lax.fori_loop(0, npairs, pbody, 0)

        for b in range(NBUF):
            wait_load(b, last_chunk)

        if rem:
            @pl.when(sid < rem)
            def _():
                ch = NTILES * base_per_tile + sid
                e0 = ch * _LANES
                pltpu.sync_copy(dst_hbm.at[pl.ds(e0, _LANES)], idx_v.at[0])
                pltpu.sync_copy(
                    h_hbm.at[pl.ds(e0, _LANES), pl.ds(col0, DC)],
                    hbuf.at[pl.ds(0, _LANES)],
                )
                pltpu.sync_copy(
                    hbuf.at[pl.ds(0, _LANES)],
                    acc_sh.at[idx_v.at[0]],
                    add=True,
                )

        plsc.subcore_barrier()
        pltpu.sync_copy(
            acc_sh.at[pl.ds(r0, rows_main)],
            out_hbm.at[pl.ds(r0, rows_main), pl.ds(col0, DC)],
        )

        @pl.when(sid == NTILES - 1)
        def _():
            pltpu.sync_copy(
                acc_sh.at[pl.ds(NTILES * rows_main, rows_tail)],
                out_hbm.at[pl.ds(NTILES * rows_main, rows_tail),
                           pl.ds(col0, DC)],
            )

    return segsum


def _make_linrelu(M, K1, K2, DO):
    BM = 1000

    def body(nodes_ref, si_ref, w1_ref, w2_ref, b_ref, o_ref):
        acc = jnp.dot(nodes_ref[...].astype(jnp.bfloat16),
                      w1_ref[...].astype(jnp.bfloat16),
                      preferred_element_type=jnp.float32)
        acc = acc + jnp.dot(si_ref[...].astype(jnp.bfloat16),
                            w2_ref[...].astype(jnp.bfloat16),
                            preferred_element_type=jnp.float32)
        o_ref[...] = jnp.maximum(acc + b_ref[...], 0.0)

    return pl.pallas_call(
        body,
        grid=(M // BM,),
        in_specs=[
            pl.BlockSpec((BM, K1), lambda i: (i, 0)),
            pl.BlockSpec((BM, K2), lambda i: (i, 0)),
            pl.BlockSpec((K1, DO), lambda i: (0, 0)),
            pl.BlockSpec((K2, DO), lambda i: (1, 0)),
            pl.BlockSpec((1, DO), lambda i: (0, 0)),
        ],
        out_specs=pl.BlockSpec((BM, DO), lambda i: (i, 0)),
        out_shape=jax.ShapeDtypeStruct((M, DO), jnp.float32),
    )


def kernel(nodes, h, edge_dst, W, b):
    N, DN = nodes.shape
    E, DE = h.shape
    DO = W.shape[1]

    zeros = jnp.zeros((624, DE // 2), jnp.float32)

    sum_inc = _make_segsum(N, E, DE)(h, edge_dst, zeros)

    out = _make_linrelu(N, DN, DE, DO)(
        nodes, sum_inc, W, W, b.reshape(1, DO)
    )
    return out


# speedup vs baseline: 1.0365x; 1.0365x over previous
import functools

import jax
import jax.numpy as jnp
from jax import lax
from jax.experimental import pallas as pl
from jax.experimental.pallas import tpu as pltpu
from jax.experimental.pallas import tpu_sc as plsc

_LANES = 128


def _make_segsum(N, E, D):
    DC = D // 2
    NCH = E // _LANES
    NTILES = 16
    base_per_tile = NCH // NTILES
    rem = NCH - base_per_tile * NTILES
    NBUF = 3
    npairs = base_per_tile // NBUF
    assert npairs * NBUF == base_per_tile
    rows_main = (N // NTILES) // 8 * 8
    rows_tail = N - rows_main * NTILES

    mesh = plsc.VectorSubcoreMesh(core_axis_name="c", subcore_axis_name="s")

    @functools.partial(
        pl.kernel,
        mesh=mesh,
        out_type=jax.ShapeDtypeStruct((N, D), jnp.float32),
        scratch_types=[
            pltpu.VMEM((NBUF, _LANES), jnp.int32),
            pltpu.VMEM((NBUF * _LANES, DC), jnp.float32),
            pltpu.VMEM_SHARED((N, DC), jnp.float32),
            pltpu.SemaphoreType.DMA,
            pltpu.SemaphoreType.DMA,
            pltpu.SemaphoreType.DMA,
        ],
    )
    def segsum(h_hbm, dst_hbm, out_hbm, idx_v, hbuf, acc_sh,
               sem0, sem1, sem2):
        cc = lax.axis_index("c")
        sid = lax.axis_index("s")
        col0 = cc * DC
        r0 = sid * rows_main
        sems = (sem0, sem1, sem2)

        chunk_base = sid * base_per_tile
        last_chunk = chunk_base + base_per_tile - 1

        def copies(b, ch):
            e0 = ch * _LANES
            return (
                pltpu.make_async_copy(
                    h_hbm.at[pl.ds(e0, _LANES), pl.ds(col0, DC)],
                    hbuf.at[pl.ds(b * _LANES, _LANES)],
                    sems[b],
                ),
                pltpu.make_async_copy(
                    dst_hbm.at[pl.ds(e0, _LANES)], idx_v.at[b], sems[b]
                ),
            )

        def start_load(b, ch):
            for c in copies(b, ch):
                c.start()

        def wait_load(b, ch):
            for c in copies(b, ch):
                c.wait()

        zvec = jnp.zeros((16,), jnp.float32)

        def zrow(r, carry):
            for j in range(DC // 16):
                hbuf[r, pl.ds(j * 16, 16)] = zvec
            return carry

        lax.fori_loop(0, _LANES, zrow, 0)
        zbuf = hbuf.at[pl.ds(0, _LANES)]
        nfull = rows_main // _LANES
        for t in range(nfull):
            pltpu.sync_copy(zbuf, acc_sh.at[pl.ds(r0 + t * _LANES, _LANES)])
        rpart = rows_main - nfull * _LANES
        if rpart:
            pltpu.sync_copy(
                hbuf.at[pl.ds(0, rpart)],
                acc_sh.at[pl.ds(r0 + nfull * _LANES, rpart)],
            )

        @pl.when(sid == NTILES - 1)
        def _():
            pltpu.sync_copy(
                hbuf.at[pl.ds(0, rows_tail)],
                acc_sh.at[pl.ds(NTILES * rows_main, rows_tail)],
            )

        plsc.subcore_barrier()

        for b in range(NBUF):
            start_load(b, chunk_base + b)

        def pbody(p, carry):
            for b in range(NBUF):
                ch = chunk_base + p * NBUF + b
                wait_load(b, ch)
                pltpu.sync_copy(
                    hbuf.at[pl.ds(b * _LANES, _LANES)],
                    acc_sh.at[idx_v.at[b]],
                    add=True,
                )
                start_load(b, jnp.minimum(ch + NBUF, last_chunk))
            return carry

        lax.fori_loop(0, npairs, pbody, 0)

        for b in range(NBUF):
            wait_load(b, last_chunk)

        if rem:
            @pl.when(sid < rem)
            def _():
                ch = NTILES * base_per_tile + sid
                e0 = ch * _LANES
                pltpu.sync_copy(dst_hbm.at[pl.ds(e0, _LANES)], idx_v.at[0])
                pltpu.sync_copy(
                    h_hbm.at[pl.ds(e0, _LANES), pl.ds(col0, DC)],
                    hbuf.at[pl.ds(0, _LANES)],
                )
                pltpu.sync_copy(
                    hbuf.at[pl.ds(0, _LANES)],
                    acc_sh.at[idx_v.at[0]],
                    add=True,
                )

        plsc.subcore_barrier()
        pltpu.sync_copy(
            acc_sh.at[pl.ds(r0, rows_main)],
            out_hbm.at[pl.ds(r0, rows_main), pl.ds(col0, DC)],
        )

        @pl.when(sid == NTILES - 1)
        def _():
            pltpu.sync_copy(
                acc_sh.at[pl.ds(NTILES * rows_main, rows_tail)],
                out_hbm.at[pl.ds(NTILES * rows_main, rows_tail),
                           pl.ds(col0, DC)],
            )

    return segsum


def _make_linrelu(M, K1, K2, DO):
    BM = 1000

    def body(nodes_ref, si_ref, w1_ref, w2_ref, b_ref, o_ref):
        acc = jnp.dot(nodes_ref[...].astype(jnp.bfloat16),
                      w1_ref[...].astype(jnp.bfloat16),
                      preferred_element_type=jnp.float32)
        acc = acc + jnp.dot(si_ref[...].astype(jnp.bfloat16),
                            w2_ref[...].astype(jnp.bfloat16),
                            preferred_element_type=jnp.float32)
        o_ref[...] = jnp.maximum(acc + b_ref[...], 0.0)

    return pl.pallas_call(
        body,
        grid=(M // BM,),
        in_specs=[
            pl.BlockSpec((BM, K1), lambda i: (i, 0)),
            pl.BlockSpec((BM, K2), lambda i: (i, 0)),
            pl.BlockSpec((K1, DO), lambda i: (0, 0)),
            pl.BlockSpec((K2, DO), lambda i: (1, 0)),
            pl.BlockSpec((1, DO), lambda i: (0, 0)),
        ],
        out_specs=pl.BlockSpec((BM, DO), lambda i: (i, 0)),
        out_shape=jax.ShapeDtypeStruct((M, DO), jnp.float32),
    )


def kernel(nodes, h, edge_dst, W, b):
    N, DN = nodes.shape
    E, DE = h.shape
    DO = W.shape[1]

    sum_inc = _make_segsum(N, E, DE)(h, edge_dst)

    out = _make_linrelu(N, DN, DE, DO)(
        nodes, sum_inc, W, W, b.reshape(1, DO)
    )
    return out


# speedup vs baseline: 1.0452x; 1.0084x over previous
import functools

import jax
import jax.numpy as jnp
from jax import lax
from jax.experimental import pallas as pl
from jax.experimental.pallas import tpu as pltpu
from jax.experimental.pallas import tpu_sc as plsc

_LANES = 128


def _make_segsum(N, E, D):
    DC = D // 2
    NCH = E // _LANES
    NTILES = 16
    base_per_tile = NCH // NTILES
    rem = NCH - base_per_tile * NTILES
    NBUF = 3
    npairs = base_per_tile // NBUF
    assert npairs * NBUF == base_per_tile
    rows_main = (N // NTILES) // 8 * 8
    rows_tail = N - rows_main * NTILES

    mesh = plsc.VectorSubcoreMesh(core_axis_name="c", subcore_axis_name="s")

    @functools.partial(
        pl.kernel,
        mesh=mesh,
        out_type=jax.ShapeDtypeStruct((N, D), jnp.float32),
        scratch_types=[
            pltpu.VMEM((NBUF, _LANES), jnp.int32),
            pltpu.VMEM((NBUF * _LANES, DC), jnp.float32),
            pltpu.VMEM_SHARED((N, DC), jnp.float32),
            pltpu.SemaphoreType.DMA,
            pltpu.SemaphoreType.DMA,
            pltpu.SemaphoreType.DMA,
        ],
    )
    def segsum(h_hbm, dst_hbm, out_hbm, idx_v, hbuf, acc_sh,
               sem0, sem1, sem2):
        cc = lax.axis_index("c")
        sid = lax.axis_index("s")
        col0 = cc * DC
        r0 = sid * rows_main
        sems = (sem0, sem1, sem2)

        chunk_base = sid * base_per_tile
        last_chunk = chunk_base + base_per_tile - 1

        def copies(b, ch):
            e0 = ch * _LANES
            return (
                pltpu.make_async_copy(
                    h_hbm.at[pl.ds(e0, _LANES), pl.ds(col0, DC)],
                    hbuf.at[pl.ds(b * _LANES, _LANES)],
                    sems[b],
                ),
                pltpu.make_async_copy(
                    dst_hbm.at[pl.ds(e0, _LANES)], idx_v.at[b], sems[b]
                ),
            )

        def start_load(b, ch):
            for c in copies(b, ch):
                c.start()

        def wait_load(b, ch):
            for c in copies(b, ch):
                c.wait()

        zvec = jnp.zeros((16,), jnp.float32)

        def zrow(r, carry):
            for j in range(DC // 16):
                hbuf[r, pl.ds(j * 16, 16)] = zvec
            return carry

        lax.fori_loop(0, _LANES, zrow, 0)
        zbuf = hbuf.at[pl.ds(0, _LANES)]
        nfull = rows_main // _LANES
        for t in range(nfull):
            pltpu.sync_copy(zbuf, acc_sh.at[pl.ds(r0 + t * _LANES, _LANES)])
        rpart = rows_main - nfull * _LANES
        if rpart:
            pltpu.sync_copy(
                hbuf.at[pl.ds(0, rpart)],
                acc_sh.at[pl.ds(r0 + nfull * _LANES, rpart)],
            )

        @pl.when(sid == NTILES - 1)
        def _():
            pltpu.sync_copy(
                hbuf.at[pl.ds(0, rows_tail)],
                acc_sh.at[pl.ds(NTILES * rows_main, rows_tail)],
            )

        plsc.subcore_barrier()

        for b in range(NBUF):
            start_load(b, chunk_base + b)

        def pbody(p, carry):
            for b in range(NBUF):
                ch = chunk_base + p * NBUF + b
                wait_load(b, ch)
                pltpu.sync_copy(
                    hbuf.at[pl.ds(b * _LANES, _LANES)],
                    acc_sh.at[idx_v.at[b]],
                    add=True,
                )
                start_load(b, jnp.minimum(ch + NBUF, last_chunk))
            return carry

        lax.fori_loop(0, npairs, pbody, 0)

        for b in range(NBUF):
            wait_load(b, last_chunk)

        if rem:
            @pl.when(sid < rem)
            def _():
                ch = NTILES * base_per_tile + sid
                e0 = ch * _LANES
                pltpu.sync_copy(dst_hbm.at[pl.ds(e0, _LANES)], idx_v.at[0])
                pltpu.sync_copy(
                    h_hbm.at[pl.ds(e0, _LANES), pl.ds(col0, DC)],
                    hbuf.at[pl.ds(0, _LANES)],
                )
                pltpu.sync_copy(
                    hbuf.at[pl.ds(0, _LANES)],
                    acc_sh.at[idx_v.at[0]],
                    add=True,
                )

        plsc.subcore_barrier()
        pltpu.sync_copy(
            acc_sh.at[pl.ds(r0, rows_main)],
            out_hbm.at[pl.ds(r0, rows_main), pl.ds(col0, DC)],
        )

        @pl.when(sid == NTILES - 1)
        def _():
            pltpu.sync_copy(
                acc_sh.at[pl.ds(NTILES * rows_main, rows_tail)],
                out_hbm.at[pl.ds(NTILES * rows_main, rows_tail),
                           pl.ds(col0, DC)],
            )

    return segsum


def _make_linrelu(M, K1, K2, DO):
    BM = 2000

    def body(nodes_ref, si_ref, w1_ref, w2_ref, b_ref, o_ref):
        acc = jnp.dot(nodes_ref[...].astype(jnp.bfloat16),
                      w1_ref[...].astype(jnp.bfloat16),
                      preferred_element_type=jnp.float32)
        acc = acc + jnp.dot(si_ref[...].astype(jnp.bfloat16),
                            w2_ref[...].astype(jnp.bfloat16),
                            preferred_element_type=jnp.float32)
        o_ref[...] = jnp.maximum(acc + b_ref[...], 0.0)

    return pl.pallas_call(
        body,
        grid=(M // BM,),
        in_specs=[
            pl.BlockSpec((BM, K1), lambda i: (i, 0)),
            pl.BlockSpec((BM, K2), lambda i: (i, 0)),
            pl.BlockSpec((K1, DO), lambda i: (0, 0)),
            pl.BlockSpec((K2, DO), lambda i: (1, 0)),
            pl.BlockSpec((1, DO), lambda i: (0, 0)),
        ],
        out_specs=pl.BlockSpec((BM, DO), lambda i: (i, 0)),
        out_shape=jax.ShapeDtypeStruct((M, DO), jnp.float32),
    )


def kernel(nodes, h, edge_dst, W, b):
    N, DN = nodes.shape
    E, DE = h.shape
    DO = W.shape[1]

    sum_inc = _make_segsum(N, E, DE)(h, edge_dst)

    out = _make_linrelu(N, DN, DE, DO)(
        nodes, sum_inc, W, W, b.reshape(1, DO)
    )
    return out


# speedup vs baseline: 1.0649x; 1.0188x over previous
import functools

import jax
import jax.numpy as jnp
from jax import lax
from jax.experimental import pallas as pl
from jax.experimental.pallas import tpu as pltpu
from jax.experimental.pallas import tpu_sc as plsc

_LANES = 128


def _make_segsum(N, E, D):
    DC = D // 2
    NCH = E // _LANES
    NTILES = 16
    base_per_tile = NCH // NTILES
    rem = NCH - base_per_tile * NTILES
    NBUF = 3
    npairs = base_per_tile // NBUF
    assert npairs * NBUF == base_per_tile
    rows_main = (N // NTILES) // 8 * 8
    rows_tail = N - rows_main * NTILES

    mesh = plsc.VectorSubcoreMesh(core_axis_name="c", subcore_axis_name="s")

    @functools.partial(
        pl.kernel,
        mesh=mesh,
        out_type=jax.ShapeDtypeStruct((N, D), jnp.float32),
        scratch_types=[
            pltpu.VMEM((NBUF, _LANES), jnp.int32),
            pltpu.VMEM((NBUF * _LANES, DC), jnp.float32),
            pltpu.VMEM_SHARED((N, DC), jnp.float32),
            pltpu.SemaphoreType.DMA,
            pltpu.SemaphoreType.DMA,
            pltpu.SemaphoreType.DMA,
        ],
    )
    def segsum(h_hbm, dst_hbm, out_hbm, idx_v, hbuf, acc_sh,
               sem0, sem1, sem2):
        cc = lax.axis_index("c")
        sid = lax.axis_index("s")
        col0 = cc * DC
        r0 = sid * rows_main
        sems = (sem0, sem1, sem2)

        chunk_base = sid * base_per_tile
        last_chunk = chunk_base + base_per_tile - 1

        def copies(b, ch):
            e0 = ch * _LANES
            return (
                pltpu.make_async_copy(
                    h_hbm.at[pl.ds(e0, _LANES), pl.ds(col0, DC)],
                    hbuf.at[pl.ds(b * _LANES, _LANES)],
                    sems[b],
                ),
                pltpu.make_async_copy(
                    dst_hbm.at[pl.ds(e0, _LANES)], idx_v.at[b], sems[b]
                ),
            )

        def start_load(b, ch):
            for c in copies(b, ch):
                c.start()

        def wait_load(b, ch):
            for c in copies(b, ch):
                c.wait()

        zvec = jnp.zeros((16,), jnp.float32)

        def zrow(r, carry):
            for j in range(DC // 16):
                hbuf[r, pl.ds(j * 16, 16)] = zvec
            return carry

        lax.fori_loop(0, _LANES, zrow, 0)
        zbuf = hbuf.at[pl.ds(0, _LANES)]
        nfull = rows_main // _LANES
        for t in range(nfull):
            pltpu.sync_copy(zbuf, acc_sh.at[pl.ds(r0 + t * _LANES, _LANES)])
        rpart = rows_main - nfull * _LANES
        if rpart:
            pltpu.sync_copy(
                hbuf.at[pl.ds(0, rpart)],
                acc_sh.at[pl.ds(r0 + nfull * _LANES, rpart)],
            )

        @pl.when(sid == NTILES - 1)
        def _():
            pltpu.sync_copy(
                hbuf.at[pl.ds(0, rows_tail)],
                acc_sh.at[pl.ds(NTILES * rows_main, rows_tail)],
            )

        plsc.subcore_barrier()

        for b in range(NBUF):
            start_load(b, chunk_base + b)

        def pbody(p, carry):
            for b in range(NBUF):
                ch = chunk_base + p * NBUF + b
                wait_load(b, ch)
                pltpu.sync_copy(
                    hbuf.at[pl.ds(b * _LANES, _LANES)],
                    acc_sh.at[idx_v.at[b]],
                    add=True,
                )
                start_load(b, jnp.minimum(ch + NBUF, last_chunk))
            return carry

        lax.fori_loop(0, npairs, pbody, 0)

        for b in range(NBUF):
            wait_load(b, last_chunk)

        if rem:
            @pl.when(sid < rem)
            def _():
                ch = NTILES * base_per_tile + sid
                e0 = ch * _LANES
                pltpu.sync_copy(dst_hbm.at[pl.ds(e0, _LANES)], idx_v.at[0])
                pltpu.sync_copy(
                    h_hbm.at[pl.ds(e0, _LANES), pl.ds(col0, DC)],
                    hbuf.at[pl.ds(0, _LANES)],
                )
                pltpu.sync_copy(
                    hbuf.at[pl.ds(0, _LANES)],
                    acc_sh.at[idx_v.at[0]],
                    add=True,
                )

        plsc.subcore_barrier()
        pltpu.sync_copy(
            acc_sh.at[pl.ds(r0, rows_main)],
            out_hbm.at[pl.ds(r0, rows_main), pl.ds(col0, DC)],
        )

        @pl.when(sid == NTILES - 1)
        def _():
            pltpu.sync_copy(
                acc_sh.at[pl.ds(NTILES * rows_main, rows_tail)],
                out_hbm.at[pl.ds(NTILES * rows_main, rows_tail),
                           pl.ds(col0, DC)],
            )

    return segsum


def _make_linrelu(M, K1, K2, DO):
    BM = 5000

    def body(nodes_ref, si_ref, w1_ref, w2_ref, b_ref, o_ref):
        acc = jnp.dot(nodes_ref[...].astype(jnp.bfloat16),
                      w1_ref[...].astype(jnp.bfloat16),
                      preferred_element_type=jnp.float32)
        acc = acc + jnp.dot(si_ref[...].astype(jnp.bfloat16),
                            w2_ref[...].astype(jnp.bfloat16),
                            preferred_element_type=jnp.float32)
        o_ref[...] = jnp.maximum(acc + b_ref[...], 0.0)

    return pl.pallas_call(
        body,
        grid=(M // BM,),
        in_specs=[
            pl.BlockSpec((BM, K1), lambda i: (i, 0)),
            pl.BlockSpec((BM, K2), lambda i: (i, 0)),
            pl.BlockSpec((K1, DO), lambda i: (0, 0)),
            pl.BlockSpec((K2, DO), lambda i: (1, 0)),
            pl.BlockSpec((1, DO), lambda i: (0, 0)),
        ],
        out_specs=pl.BlockSpec((BM, DO), lambda i: (i, 0)),
        out_shape=jax.ShapeDtypeStruct((M, DO), jnp.float32),
    )


def kernel(nodes, h, edge_dst, W, b):
    N, DN = nodes.shape
    E, DE = h.shape
    DO = W.shape[1]

    sum_inc = _make_segsum(N, E, DE)(h, edge_dst)
    out = _make_linrelu(N, DN, DE, DO)(
        nodes, sum_inc, W, W, b.reshape(1, DO)
    )
    return out
